# baseline (speedup 1.0000x reference)
import functools

import jax
import jax.numpy as jnp
from jax import lax
from jax.experimental import pallas as pl
from jax.experimental.pallas import tpu as pltpu
from jax.experimental.pallas import tpu_sc as plsc

B, H, W = 16, 384, 384
HW = H * W
N = B * HW
NC, NS, L = 2, 16, 16
NW = NC * NS
PER_W = N // NW
S = 4096
NCHUNK = PER_W // S
VPC = S // L

SB, SCH = 3 * HW, HW
GRP = 9216
NGRP = PER_W // GRP
WINW = HW // 2
WPAD = 8


@functools.cache
def _build_sampler():
  mesh = plsc.VectorSubcoreMesh(
      core_axis_name="c", subcore_axis_name="s", num_cores=NC, num_subcores=NS
  )

  @functools.partial(
      pl.kernel,
      out_type=(
          jax.ShapeDtypeStruct((N,), jnp.float32),
          jax.ShapeDtypeStruct((N // 2,), jnp.int32),
      ),
      mesh=mesh,
      compiler_params=pltpu.CompilerParams(needs_layout_passes=False),
      scratch_types=[
          [pltpu.VMEM((3072,), jnp.float32) for _ in range(3)],
          pltpu.VMEM((GRP + 16,), jnp.float32),
          pltpu.VMEM((GRP // 2,), jnp.int32),
          pltpu.VMEM((WINW + WPAD,), jnp.int32),
          [pltpu.VMEM((2 * S,), jnp.float32) for _ in range(2)],
          [pltpu.VMEM((S,), jnp.float32) for _ in range(2)],
          pltpu.SemaphoreType.DMA,
          [pltpu.SemaphoreType.DMA, pltpu.SemaphoreType.DMA],
          [pltpu.SemaphoreType.DMA, pltpu.SemaphoreType.DMA],
      ],
  )
  def _sampler(img_phys, cxy_h, out_h, pk_h, strips, floc, pbuf, win,
               cxy_v, o_v, sem_b, sem_c, sem_o):
    wid = lax.axis_index("c") * NS + lax.axis_index("s")
    base = wid * PER_W
    b = wid // 2
    iota = lax.iota(jnp.int32, L)
    i2 = iota * 2
    i3 = iota * 3

    def rne_hi(v):
      u = plsc.bitcast(v, jnp.uint32)
      return (u + 0x7FFF + ((u >> 16) & 1)) >> 16

    def group(q, carry):
      grp = base + q * GRP
      gno = wid * NGRP + q
      waits = []
      for ch in range(3):
        src = (gno // 48) * SB + ch * SCH + (gno % 48) * 3072
        waits.append(pltpu.async_copy(
            img_phys.at[pl.ds(pl.multiple_of(src, 1024), 3072)],
            strips[ch], sem_b))
      for cp in waits:
        cp.wait()

      for ch in range(3):
        @plsc.parallel_loop(0, 3072 // L, unroll=2)
        def de_body(v, ch=ch):
          m0 = v * L
          p0 = ((m0 % 1024) // 128) * 1152 + (m0 // 1024) * 384 + 3 * (m0 % 128) + ch
          vals = strips[ch][pl.ds(m0, L)]
          plsc.store_scatter(floc, [p0 + i3], vals)

      @plsc.parallel_loop(0, GRP // 2 // L, unroll=2)
      def pk_body(j):
        ev = plsc.load_gather(floc, [j * 32 + i2])
        od = plsc.load_gather(floc, [j * 32 + 1 + i2])
        w = rne_hi(ev) | (rne_hi(od) << 16)
        pbuf[pl.ds(j * L, L)] = plsc.bitcast(w, jnp.int32)
      grp2 = wid * (PER_W // 2) + q * (GRP // 2)
      pltpu.sync_copy(pbuf, pk_h.at[pl.ds(pl.multiple_of(grp2, 8), GRP // 2)])
      return carry

    lax.fori_loop(0, NGRP, group, 0)
    plsc.subcore_barrier()

    wb0 = pl.multiple_of(b * WINW, 8)
    pltpu.sync_copy(pk_h.at[pl.ds(wb0, WINW)], win.at[pl.ds(0, WINW)])
    pltpu.sync_copy(pk_h.at[pl.ds(wb0, WPAD)], win.at[pl.ds(WINW, WPAD)])

    def load(c, k):
      return pltpu.async_copy(
          cxy_h.at[pl.ds(2 * (base + c * S), 2 * S)], cxy_v[k], sem_c[k])

    def wait_load(k):
      pltpu.make_async_copy(
          cxy_h.at[pl.ds(0, 2 * S)], cxy_v[k], sem_c[k]).wait()

    def lohalf(w):
      return plsc.bitcast(w << 16, jnp.float32)

    def hihalf(w):
      return plsc.bitcast(w & -65536, jnp.float32)

    def wait_out(k):
      pltpu.make_async_copy(o_v[k], out_h.at[pl.ds(0, S)], sem_o[k]).wait()

    def chunk(c, k, wait=True):
      if wait:
        wait_out(k)

      @plsc.parallel_loop(0, VPC, unroll=4)
      def body(i):
        qo = (i // 8) * 256 + (i % 8) * L
        sl = pl.ds(i * L, L)
        cx = cxy_v[k][pl.ds(qo, L)]
        cy = cxy_v[k][pl.ds(qo + 128, L)]
        x0 = cx.astype(jnp.int32)
        y0 = cy.astype(jnp.int32)
        wx1 = cx - x0.astype(jnp.float32)
        wy1 = cy - y0.astype(jnp.float32)
        wa = jnp.where(y0 < H - 1, 1.0 - wy1, 0.0)
        gt = jnp.minimum(y0, H - 2) * W + x0
        par = gt & 1
        pm = par > 0
        wt = gt >> 1
        a0 = plsc.load_gather(win, [wt])
        a1 = plsc.load_gather(win, [wt + par])
        t0 = jnp.where(pm, hihalf(a0), lohalf(a0))
        t1 = jnp.where(pm, lohalf(a1), hihalf(a1))
        wbt = wt + (W // 2)
        b0 = plsc.load_gather(win, [wbt])
        b1 = plsc.load_gather(win, [wbt + par])
        u0 = jnp.where(pm, hihalf(b0), lohalf(b0))
        u1 = jnp.where(pm, lohalf(b1), hihalf(b1))
        wx0 = 1.0 - wx1
        o_v[k][sl] = (wa * (wx0 * t0 + wx1 * t1)
                      + (1.0 - wa) * (wx0 * u0 + wx1 * u1))
      pltpu.async_copy(o_v[k], out_h.at[pl.ds(base + c * S, S)], sem_o[k])

    load(0, 0)
    wait_load(0)
    load(1, 1)
    chunk(0, 0, wait=False)
    wait_load(1)
    load(2, 0)
    chunk(1, 1, wait=False)

    def steady(j, carry):
      for par in range(2):
        c = 2 * j + 2 + par
        k = par
        wait_load(k)
        load(c + 1, 1 - k)
        chunk(c, k)
      return carry

    lax.fori_loop(0, NCHUNK // 2 - 2, steady, 0)
    wait_load(0)
    load(NCHUNK - 1, 1)
    chunk(NCHUNK - 2, 0)
    wait_load(1)
    chunk(NCHUNK - 1, 1)
    wait_out(0)
    wait_out(1)

  return _sampler


def kernel(imgs, coords):
  img_phys = imgs.reshape(16, 48, 8, 3, 128, 3).transpose(0, 5, 1, 3, 2, 4)
  img_phys = img_phys.reshape(-1)
  cxy = coords.reshape(16, 1152, 128, 2).transpose(0, 1, 3, 2).reshape(-1)
  out, _ = _build_sampler()(img_phys, cxy)
  return out.reshape(B, H, W, 1)


# speedup vs baseline: 1.2022x; 1.2022x over previous
import functools

import jax
import jax.numpy as jnp
from jax import lax
from jax.experimental import pallas as pl
from jax.experimental.pallas import tpu as pltpu
from jax.experimental.pallas import tpu_sc as plsc

B, H, W = 16, 384, 384
HW = H * W
N = B * HW
NC, NS, L = 2, 16, 16
NW = NC * NS
PER_W = N // NW
S = 4096
NCHUNK = PER_W // S
VPC = S // L

SB, SCH = 3 * HW, HW
GRP = 9216
NGRP = PER_W // GRP
WINW = HW // 2
WPAD = 8


@functools.cache
def _build_sampler():
  mesh = plsc.VectorSubcoreMesh(
      core_axis_name="c", subcore_axis_name="s", num_cores=NC, num_subcores=NS
  )

  @functools.partial(
      pl.kernel,
      out_type=(
          jax.ShapeDtypeStruct((N,), jnp.float32),
          jax.ShapeDtypeStruct((N // 2,), jnp.int32),
      ),
      mesh=mesh,
      compiler_params=pltpu.CompilerParams(needs_layout_passes=False),
      scratch_types=[
          [[pltpu.VMEM((3072,), jnp.float32) for _ in range(3)]
           for _ in range(2)],
          pltpu.VMEM((GRP + 16,), jnp.float32),
          pltpu.VMEM((GRP // 2,), jnp.int32),
          pltpu.VMEM((WINW + WPAD,), jnp.int32),
          [pltpu.VMEM((2 * S,), jnp.float32) for _ in range(2)],
          [pltpu.VMEM((S,), jnp.float32) for _ in range(2)],
          [pltpu.SemaphoreType.DMA, pltpu.SemaphoreType.DMA],
          [pltpu.SemaphoreType.DMA, pltpu.SemaphoreType.DMA],
          [pltpu.SemaphoreType.DMA, pltpu.SemaphoreType.DMA],
      ],
  )
  def _sampler(img_phys, cxy_h, out_h, pk_h, strips, floc, pbuf, win,
               cxy_v, o_v, sem_b, sem_c, sem_o):
    wid = lax.axis_index("c") * NS + lax.axis_index("s")
    base = wid * PER_W
    b = wid // 2
    iota = lax.iota(jnp.int32, L)
    i2 = iota * 2
    i3 = iota * 3

    def rne_hi(v):
      u = plsc.bitcast(v, jnp.uint32)
      return (u + 0x7FFF + ((u >> 16) & 1)) >> 16

    def issue_strips(q, kk):
      gno = wid * NGRP + q
      for ch in range(3):
        src = (gno // 48) * SB + ch * SCH + (gno % 48) * 3072
        pltpu.async_copy(
            img_phys.at[pl.ds(pl.multiple_of(src, 1024), 3072)],
            strips[kk][ch], sem_b[kk])

    def wait_strips(kk):
      for ch in range(3):
        pltpu.make_async_copy(
            img_phys.at[pl.ds(0, 3072)], strips[kk][ch], sem_b[kk]).wait()

    def group(q, kk):
      wait_strips(kk)
      issue_strips(q + 1, 1 - kk)

      for ch in range(3):
        @plsc.parallel_loop(0, 3072 // L, unroll=2)
        def de_body(v, ch=ch):
          m0 = v * L
          p0 = ((m0 % 1024) // 128) * 1152 + (m0 // 1024) * 384 + 3 * (m0 % 128) + ch
          vals = strips[kk][ch][pl.ds(m0, L)]
          plsc.store_scatter(floc, [p0 + i3], vals)

      @plsc.parallel_loop(0, GRP // 2 // L, unroll=2)
      def pk_body(j):
        ev = plsc.load_gather(floc, [j * 32 + i2])
        od = plsc.load_gather(floc, [j * 32 + 1 + i2])
        w = rne_hi(ev) | (rne_hi(od) << 16)
        pbuf[pl.ds(j * L, L)] = plsc.bitcast(w, jnp.int32)
      grp2 = wid * (PER_W // 2) + q * (GRP // 2)
      pltpu.sync_copy(pbuf, pk_h.at[pl.ds(pl.multiple_of(grp2, 8), GRP // 2)])

    issue_strips(0, 0)

    def bgroup(j, carry):
      group(2 * j, 0)
      group(2 * j + 1, 1)
      return carry

    lax.fori_loop(0, NGRP // 2, bgroup, 0)
    wait_strips(0)
    plsc.subcore_barrier()

    wb0 = pl.multiple_of(b * WINW, 8)
    pltpu.sync_copy(pk_h.at[pl.ds(wb0, WINW)], win.at[pl.ds(0, WINW)])
    pltpu.sync_copy(pk_h.at[pl.ds(wb0, WPAD)], win.at[pl.ds(WINW, WPAD)])

    def load(c, k):
      return pltpu.async_copy(
          cxy_h.at[pl.ds(2 * (base + c * S), 2 * S)], cxy_v[k], sem_c[k])

    def wait_load(k):
      pltpu.make_async_copy(
          cxy_h.at[pl.ds(0, 2 * S)], cxy_v[k], sem_c[k]).wait()

    def lohalf(w):
      return plsc.bitcast(w << 16, jnp.float32)

    def hihalf(w):
      return plsc.bitcast(w & -65536, jnp.float32)

    def wait_out(k):
      pltpu.make_async_copy(o_v[k], out_h.at[pl.ds(0, S)], sem_o[k]).wait()

    def chunk(c, k, wait=True):
      if wait:
        wait_out(k)

      @plsc.parallel_loop(0, VPC, unroll=4)
      def body(i):
        qo = (i // 8) * 256 + (i % 8) * L
        sl = pl.ds(i * L, L)
        cx = cxy_v[k][pl.ds(qo, L)]
        cy = cxy_v[k][pl.ds(qo + 128, L)]
        x0 = cx.astype(jnp.int32)
        y0 = cy.astype(jnp.int32)
        wx1 = cx - x0.astype(jnp.float32)
        wy1 = cy - y0.astype(jnp.float32)
        wa = jnp.where(y0 < H - 1, 1.0 - wy1, 0.0)
        gt = jnp.minimum(y0, H - 2) * W + x0
        par = gt & 1
        pm = par > 0
        wt = gt >> 1
        a0 = plsc.load_gather(win, [wt])
        a1 = plsc.load_gather(win, [wt + par])
        t0 = jnp.where(pm, hihalf(a0), lohalf(a0))
        t1 = jnp.where(pm, lohalf(a1), hihalf(a1))
        wbt = wt + (W // 2)
        b0 = plsc.load_gather(win, [wbt])
        b1 = plsc.load_gather(win, [wbt + par])
        u0 = jnp.where(pm, hihalf(b0), lohalf(b0))
        u1 = jnp.where(pm, lohalf(b1), hihalf(b1))
        wx0 = 1.0 - wx1
        o_v[k][sl] = (wa * (wx0 * t0 + wx1 * t1)
                      + (1.0 - wa) * (wx0 * u0 + wx1 * u1))
      pltpu.async_copy(o_v[k], out_h.at[pl.ds(base + c * S, S)], sem_o[k])

    load(0, 0)
    wait_load(0)
    load(1, 1)
    chunk(0, 0, wait=False)
    wait_load(1)
    load(2, 0)
    chunk(1, 1, wait=False)

    def steady(j, carry):
      for par in range(2):
        c = 2 * j + 2 + par
        k = par
        wait_load(k)
        load(c + 1, 1 - k)
        chunk(c, k)
      return carry

    lax.fori_loop(0, NCHUNK // 2 - 2, steady, 0)
    wait_load(0)
    load(NCHUNK - 1, 1)
    chunk(NCHUNK - 2, 0)
    wait_load(1)
    chunk(NCHUNK - 1, 1)
    wait_out(0)
    wait_out(1)

  return _sampler


def kernel(imgs, coords):
  img_phys = imgs.reshape(16, 48, 8, 3, 128, 3).transpose(0, 5, 1, 3, 2, 4)
  img_phys = img_phys.reshape(-1)
  cxy = coords.reshape(16, 384, 3, 128, 2).transpose(0, 1, 2, 4, 3).reshape(-1)
  out, _ = _build_sampler()(img_phys, cxy)
  return out.reshape(B, H, W, 1)


# speedup vs baseline: 1.2191x; 1.0140x over previous
import functools

import jax
import jax.numpy as jnp
from jax import lax
from jax.experimental import pallas as pl
from jax.experimental.pallas import tpu as pltpu
from jax.experimental.pallas import tpu_sc as plsc

B, H, W = 16, 384, 384
HW = H * W
N = B * HW
NC, NS, L = 2, 16, 16
NW = NC * NS
PER_W = N // NW
S = 4096
NCHUNK = PER_W // S
VPC = S // L

SB, SCH = 3 * HW, HW
GRP = 9216
NGRP = PER_W // GRP
WINW = HW // 2
WPAD = 8


@functools.cache
def _build_sampler():
  mesh = plsc.VectorSubcoreMesh(
      core_axis_name="c", subcore_axis_name="s", num_cores=NC, num_subcores=NS
  )

  @functools.partial(
      pl.kernel,
      out_type=(
          jax.ShapeDtypeStruct((N,), jnp.float32),
          jax.ShapeDtypeStruct((N // 2,), jnp.int32),
      ),
      mesh=mesh,
      compiler_params=pltpu.CompilerParams(needs_layout_passes=False),
      scratch_types=[
          [[pltpu.VMEM((3072,), jnp.float32) for _ in range(3)]
           for _ in range(2)],
          pltpu.VMEM((GRP + 16,), jnp.float32),
          pltpu.VMEM((GRP // 2,), jnp.int32),
          pltpu.VMEM((WINW + WPAD,), jnp.int32),
          [pltpu.VMEM((2 * S,), jnp.float32) for _ in range(2)],
          [pltpu.VMEM((S,), jnp.float32) for _ in range(2)],
          [pltpu.SemaphoreType.DMA, pltpu.SemaphoreType.DMA],
          [pltpu.SemaphoreType.DMA, pltpu.SemaphoreType.DMA],
          [pltpu.SemaphoreType.DMA, pltpu.SemaphoreType.DMA],
      ],
  )
  def _sampler(img_phys, cxy_h, out_h, pk_h, strips, floc, pbuf, win,
               cxy_v, o_v, sem_b, sem_c, sem_o):
    wid = lax.axis_index("c") * NS + lax.axis_index("s")
    base = wid * PER_W
    b = wid // 2
    iota = lax.iota(jnp.int32, L)
    i2 = iota * 2
    i3 = iota * 3

    def rne_hi(v):
      u = plsc.bitcast(v, jnp.uint32)
      return (u + 0x7FFF + ((u >> 16) & 1)) >> 16

    def issue_strips(q, kk):
      gno = wid * NGRP + q
      for ch in range(3):
        src = (gno // 48) * SB + ch * SCH + (gno % 48) * 3072
        pltpu.async_copy(
            img_phys.at[pl.ds(pl.multiple_of(src, 1024), 3072)],
            strips[kk][ch], sem_b[kk])

    def wait_strips(kk):
      for ch in range(3):
        pltpu.make_async_copy(
            img_phys.at[pl.ds(0, 3072)], strips[kk][ch], sem_b[kk]).wait()

    def group(q, kk):
      wait_strips(kk)
      issue_strips(q + 1, 1 - kk)

      for ch in range(3):
        @plsc.parallel_loop(0, 3072 // L, unroll=4)
        def de_body(v, ch=ch):
          m0 = v * L
          p0 = ((m0 % 1024) // 128) * 1152 + (m0 // 1024) * 384 + 3 * (m0 % 128) + ch
          vals = strips[kk][ch][pl.ds(m0, L)]
          plsc.store_scatter(floc, [p0 + i3], vals)

      @plsc.parallel_loop(0, GRP // 2 // L, unroll=4)
      def pk_body(j):
        ev = plsc.load_gather(floc, [j * 32 + i2])
        od = plsc.load_gather(floc, [j * 32 + 1 + i2])
        w = rne_hi(ev) | (rne_hi(od) << 16)
        pbuf[pl.ds(j * L, L)] = plsc.bitcast(w, jnp.int32)
      grp2 = wid * (PER_W // 2) + q * (GRP // 2)
      pltpu.sync_copy(pbuf, pk_h.at[pl.ds(pl.multiple_of(grp2, 8), GRP // 2)])

    issue_strips(0, 0)

    def bgroup(j, carry):
      group(2 * j, 0)
      group(2 * j + 1, 1)
      return carry

    lax.fori_loop(0, NGRP // 2, bgroup, 0)
    wait_strips(0)
    plsc.subcore_barrier()

    wb0 = pl.multiple_of(b * WINW, 8)
    pltpu.sync_copy(pk_h.at[pl.ds(wb0, WINW)], win.at[pl.ds(0, WINW)])
    pltpu.sync_copy(pk_h.at[pl.ds(wb0, WPAD)], win.at[pl.ds(WINW, WPAD)])

    def load(c, k):
      return pltpu.async_copy(
          cxy_h.at[pl.ds(2 * (base + c * S), 2 * S)], cxy_v[k], sem_c[k])

    def wait_load(k):
      pltpu.make_async_copy(
          cxy_h.at[pl.ds(0, 2 * S)], cxy_v[k], sem_c[k]).wait()

    def lohalf(w):
      return plsc.bitcast(w << 16, jnp.float32)

    def hihalf(w):
      return plsc.bitcast(w & -65536, jnp.float32)

    def wait_out(k):
      pltpu.make_async_copy(o_v[k], out_h.at[pl.ds(0, S)], sem_o[k]).wait()

    def chunk(c, k, wait=True):
      if wait:
        wait_out(k)

      @plsc.parallel_loop(0, VPC, unroll=4)
      def body(i):
        qo = (i // 8) * 256 + (i % 8) * L
        sl = pl.ds(i * L, L)
        cx = cxy_v[k][pl.ds(qo, L)]
        cy = cxy_v[k][pl.ds(qo + 128, L)]
        x0 = cx.astype(jnp.int32)
        y0 = cy.astype(jnp.int32)
        wx1 = cx - x0.astype(jnp.float32)
        wy1 = cy - y0.astype(jnp.float32)
        wa = jnp.where(y0 < H - 1, 1.0 - wy1, 0.0)
        gt = jnp.minimum(y0, H - 2) * W + x0
        par = gt & 1
        pm = par > 0
        wt = gt >> 1
        a0 = plsc.load_gather(win, [wt])
        a1 = plsc.load_gather(win, [wt + par])
        t0 = jnp.where(pm, hihalf(a0), lohalf(a0))
        t1 = jnp.where(pm, lohalf(a1), hihalf(a1))
        wbt = wt + (W // 2)
        b0 = plsc.load_gather(win, [wbt])
        b1 = plsc.load_gather(win, [wbt + par])
        u0 = jnp.where(pm, hihalf(b0), lohalf(b0))
        u1 = jnp.where(pm, lohalf(b1), hihalf(b1))
        wx0 = 1.0 - wx1
        o_v[k][sl] = (wa * (wx0 * t0 + wx1 * t1)
                      + (1.0 - wa) * (wx0 * u0 + wx1 * u1))
      pltpu.async_copy(o_v[k], out_h.at[pl.ds(base + c * S, S)], sem_o[k])

    load(0, 0)
    wait_load(0)
    load(1, 1)
    chunk(0, 0, wait=False)
    wait_load(1)
    load(2, 0)
    chunk(1, 1, wait=False)

    def steady(j, carry):
      for par in range(2):
        c = 2 * j + 2 + par
        k = par
        wait_load(k)
        load(c + 1, 1 - k)
        chunk(c, k)
      return carry

    lax.fori_loop(0, NCHUNK // 2 - 2, steady, 0)
    wait_load(0)
    load(NCHUNK - 1, 1)
    chunk(NCHUNK - 2, 0)
    wait_load(1)
    chunk(NCHUNK - 1, 1)
    wait_out(0)
    wait_out(1)

  return _sampler


def kernel(imgs, coords):
  img_phys = imgs.reshape(16, 48, 8, 3, 128, 3).transpose(0, 5, 1, 3, 2, 4)
  img_phys = img_phys.reshape(-1)
  cxy = coords.reshape(16, 384, 3, 128, 2).transpose(0, 1, 2, 4, 3).reshape(-1)
  out, _ = _build_sampler()(img_phys, cxy)
  return out.reshape(B, H, W, 1)
